# Initial kernel scaffold; baseline (speedup 1.0000x reference)
import jax
import jax.numpy as jnp
from jax.experimental import pallas as pl


def kernel(indices, prototypes):
    raise NotImplementedError("write your pallas kernel here")


# speedup vs baseline: 1.8450x; 1.8450x over previous
import functools

import jax
import jax.numpy as jnp
from jax import lax
from jax.experimental import pallas as pl
from jax.experimental.pallas import tpu as pltpu
from jax.experimental.pallas import tpu_sc as plsc

NUM_PROTOTYPES = 1000
EMBED_DIM = 512
BATCH = 16384

_INFO = plsc.get_sparse_core_info()
_NC, _NS = _INFO.num_cores, _INFO.num_subcores
_NW = _NC * _NS
_B_PER_W = BATCH // _NW
_CHUNK = 64
_N_CHUNK = _B_PER_W // _CHUNK


def _make_gather():
  mesh = plsc.VectorSubcoreMesh(core_axis_name="c", subcore_axis_name="s")

  @functools.partial(
      pl.kernel,
      mesh=mesh,
      out_type=jax.ShapeDtypeStruct((BATCH, EMBED_DIM), jnp.float32),
      scratch_types=[
          pltpu.VMEM((_N_CHUNK, _CHUNK), jnp.int32),
          pltpu.VMEM((_CHUNK, EMBED_DIM), jnp.float32),
          pltpu.VMEM((_CHUNK, EMBED_DIM), jnp.float32),
          pltpu.SemaphoreType.DMA,
          pltpu.SemaphoreType.DMA,
          pltpu.SemaphoreType.DMA,
          pltpu.SemaphoreType.DMA,
      ],
  )
  def gather_kernel(table_hbm, idx_hbm, out_hbm, idx_v, buf0, buf1,
                    g0, g1, w0, w1):
    wid = lax.axis_index("c") * _NS + lax.axis_index("s")
    base = wid * _B_PER_W
    pltpu.sync_copy(idx_hbm.at[wid], idx_v)

    bufs = (buf0, buf1)
    gsems = (g0, g1)
    wsems = (w0, w1)

    pltpu.async_copy(table_hbm.at[idx_v.at[0]], buf0, g0)
    pltpu.async_copy(table_hbm.at[idx_v.at[1]], buf1, g1)

    for c in range(_N_CHUNK):
      b = c % 2
      pltpu.make_async_copy(table_hbm.at[idx_v.at[c]], bufs[b], gsems[b]).wait()
      out_slice = out_hbm.at[pl.ds(base + c * _CHUNK, _CHUNK)]
      pltpu.async_copy(bufs[b], out_slice, wsems[b])
      if c + 2 < _N_CHUNK:
        pltpu.make_async_copy(bufs[b], out_slice, wsems[b]).wait()
        pltpu.async_copy(table_hbm.at[idx_v.at[c + 2]], bufs[b], gsems[b])

    for c in (_N_CHUNK - 2, _N_CHUNK - 1):
      b = c % 2
      out_slice = out_hbm.at[pl.ds(base + c * _CHUNK, _CHUNK)]
      pltpu.make_async_copy(bufs[b], out_slice, wsems[b]).wait()

  return gather_kernel


_gather = _make_gather()


@jax.jit
def kernel(indices, prototypes):
  idx = indices.astype(jnp.int32).reshape(_NW, _N_CHUNK, _CHUNK)
  return _gather(prototypes, idx)
